# baseline (speedup 1.0000x reference)
import functools

import jax
import jax.numpy as jnp
from jax import lax
from jax.experimental import pallas as pl
from jax.experimental.pallas import tpu as pltpu
from jax.experimental.pallas import tpu_sc as plsc

N_NODES = 16384
NUM_ARGS = 26
DIM = 64
LANES = 16


def _make_sc_pool():
    info = plsc.get_sparse_core_info()
    nc, ns = info.num_cores, info.num_subcores
    nw = nc * ns
    n_per_w = N_NODES // nw

    mesh = plsc.VectorSubcoreMesh(core_axis_name="c", subcore_axis_name="s")

    @functools.partial(
        pl.kernel,
        out_type=jax.ShapeDtypeStruct((N_NODES, DIM), jnp.float32),
        mesh=mesh,
        scratch_types=[
            pltpu.VMEM((n_per_w,), jnp.int32),
            pltpu.VMEM((n_per_w,), jnp.int32),
            pltpu.VMEM((n_per_w, DIM), jnp.float32),
            pltpu.VMEM((n_per_w, DIM), jnp.float32),
            pltpu.VMEM((n_per_w, DIM), jnp.float32),
            pltpu.SemaphoreType.DMA,
            pltpu.SemaphoreType.DMA,
        ],
        compiler_params=pltpu.CompilerParams(use_tc_tiling_on_sc=False),
    )
    def sc_pool(idx_hbm, table_hbm, out_hbm,
                idx0, idx1, rows0, rows1, acc, sem0, sem1):
        wid = lax.axis_index("s") * nc + lax.axis_index("c")
        base = wid * n_per_w
        idx_bufs = (idx0, idx1)
        row_bufs = (rows0, rows1)
        sems = (sem0, sem1)

        def load_idx(j, slot):
            pltpu.sync_copy(
                idx_hbm.at[pl.ds(j * N_NODES + base, n_per_w)],
                idx_bufs[slot],
            )

        def gather(slot):
            return pltpu.make_async_copy(
                table_hbm.at[idx_bufs[slot]], row_bufs[slot], sems[slot]
            )

        def accumulate(rbuf, first):
            def body(r, carry):
                for c in range(DIM // LANES):
                    sl = pl.ds(c * LANES, LANES)
                    v = rbuf[r, sl]
                    if first:
                        acc[r, sl] = v
                    else:
                        plsc.addupdate(acc.at[r, sl], v)
                return carry
            lax.fori_loop(0, n_per_w, body, 0, unroll=4)

        load_idx(0, 0)
        gather(0).start()
        for j in range(NUM_ARGS):
            slot = j % 2
            if j + 1 < NUM_ARGS:
                load_idx(j + 1, 1 - slot)
                gather(1 - slot).start()
            gather(slot).wait()
            accumulate(row_bufs[slot], first=(j == 0))
        pltpu.sync_copy(acc, out_hbm.at[pl.ds(base, n_per_w)])

    return sc_pool


_sc_pool = _make_sc_pool()


def _tc_head(pooled, lens2d, W_mu, b_mu2d, W_sigma, b_sigma2d):
    blk = 2048
    grid = (N_NODES // blk,)

    def body(p_ref, l_ref, wm_ref, bm_ref, ws_ref, bs_ref, mu_ref, ls_ref):
        h = jnp.tanh(p_ref[...] / l_ref[...])
        mu_ref[...] = (
            jnp.dot(h, wm_ref[...], preferred_element_type=jnp.float32)
            + bm_ref[...]
        )
        ls_ref[...] = (
            jnp.dot(h, ws_ref[...], preferred_element_type=jnp.float32)
            + bs_ref[...]
        )

    return pl.pallas_call(
        body,
        grid=grid,
        in_specs=[
            pl.BlockSpec((blk, DIM), lambda i: (i, 0)),
            pl.BlockSpec((blk, 1), lambda i: (i, 0)),
            pl.BlockSpec((DIM, DIM), lambda i: (0, 0)),
            pl.BlockSpec((1, DIM), lambda i: (0, 0)),
            pl.BlockSpec((DIM, DIM), lambda i: (0, 0)),
            pl.BlockSpec((1, DIM), lambda i: (0, 0)),
        ],
        out_specs=[
            pl.BlockSpec((blk, DIM), lambda i: (i, 0)),
            pl.BlockSpec((blk, DIM), lambda i: (i, 0)),
        ],
        out_shape=[
            jax.ShapeDtypeStruct((N_NODES, DIM), jnp.float32),
            jax.ShapeDtypeStruct((N_NODES, DIM), jnp.float32),
        ],
    )(pooled, lens2d, W_mu, b_mu2d, W_sigma, b_sigma2d)


def kernel(pred_func_nodes_ctxt_predargs, pred_func_nodes_ctxt_predargs_len,
           device, train_mode, table, W_mu, b_mu, W_sigma, b_sigma):
    idx = pred_func_nodes_ctxt_predargs
    idx_flat = (idx.T + jnp.int32(1)).reshape(-1)
    pooled = _sc_pool(idx_flat, table)
    lens2d = pred_func_nodes_ctxt_predargs_len.reshape(N_NODES, 1)
    mu, log_sigma2 = _tc_head(
        pooled, lens2d, W_mu, b_mu.reshape(1, DIM), W_sigma,
        b_sigma.reshape(1, DIM),
    )
    return mu[None], log_sigma2[None]
